# baseline (speedup 1.0000x reference)
import functools

import jax
import jax.numpy as jnp
from jax.experimental import pallas as pl

_NUM_CODES = 8192
_LATENT_DIM = 64
_BETA = 0.25
_BT = 512


def _vq_block(z_ref, c_ref, zq_ref, idx_ref, loss_ref):
    i = pl.program_id(0)
    zb = z_ref[...]
    cb = c_ref[...]
    z2 = jnp.sum(zb * zb, axis=1, keepdims=True)
    c2 = jnp.sum(cb * cb, axis=1)
    m2 = jnp.dot(zb + zb, cb.T, preferred_element_type=jnp.float32)
    dist = (z2 + c2[None, :]) - m2
    minval = jnp.min(dist, axis=1, keepdims=True)
    col = jax.lax.broadcasted_iota(jnp.int32, (_BT, _NUM_CODES), 1)
    idx = jnp.min(jnp.where(dist == minval, col, _NUM_CODES), axis=1)
    idx_ref[...] = idx.reshape(1, 1, _BT)
    onehot = (col == idx[:, None]).astype(jnp.bfloat16)
    zq = jnp.dot(onehot, cb.astype(jnp.bfloat16),
                 preferred_element_type=jnp.float32)
    zq_ref[...] = zq
    diff = zq - zb
    part = jnp.sum(diff * diff)

    @pl.when(i == 0)
    def _():
        loss_ref[...] = jnp.zeros((1, 1), jnp.float32)

    loss_ref[...] += jnp.reshape(part, (1, 1))


@functools.partial(jax.jit, static_argnames=())
def kernel(z, codebook):
    B, C, H, W = z.shape
    T = B * H * W
    nb = T // _BT
    zp = jnp.transpose(z, (0, 2, 3, 1))
    z_flat = zp.reshape(T, _LATENT_DIM)

    zq, idx3, loss_sum = pl.pallas_call(
        _vq_block,
        grid=(nb,),
        in_specs=[
            pl.BlockSpec((_BT, _LATENT_DIM), lambda i: (i, 0)),
            pl.BlockSpec((_NUM_CODES, _LATENT_DIM), lambda i: (0, 0)),
        ],
        out_specs=[
            pl.BlockSpec((_BT, _LATENT_DIM), lambda i: (i, 0)),
            pl.BlockSpec((1, 1, _BT), lambda i: (i, 0, 0)),
            pl.BlockSpec((1, 1), lambda i: (0, 0)),
        ],
        out_shape=[
            jax.ShapeDtypeStruct((T, _LATENT_DIM), jnp.float32),
            jax.ShapeDtypeStruct((nb, 1, _BT), jnp.int32),
            jax.ShapeDtypeStruct((1, 1), jnp.float32),
        ],
    )(z_flat, codebook)

    out = zq.reshape(B, H, W, C).transpose(0, 3, 1, 2)
    indices = idx3.reshape(T)
    loss = (1.0 + _BETA) * (loss_sum[0, 0] / (T * _LATENT_DIM))
    return out, indices, loss


# speedup vs baseline: 1.3278x; 1.3278x over previous
import functools

import jax
import jax.numpy as jnp
from jax.experimental import pallas as pl

_NUM_CODES = 8192
_LATENT_DIM = 64
_BETA = 0.25
_BT = 256


def _vq_block(z_ref, c_ref, cb16_ref, c2_ref, zq_ref, idx_ref, loss_ref):
    i = pl.program_id(0)
    zb = z_ref[...]
    cb = c_ref[...]
    z2 = jnp.sum(zb * zb, axis=1, keepdims=True)
    m2 = jnp.dot(zb + zb, cb.T, preferred_element_type=jnp.float32)
    dist = (z2 + c2_ref[...]) - m2
    minval = jnp.min(dist, axis=1, keepdims=True)
    col = jax.lax.broadcasted_iota(jnp.int32, (_BT, _NUM_CODES), 1)
    idx = jnp.min(jnp.where(dist == minval, col, _NUM_CODES), axis=1)
    idx_ref[...] = idx.reshape(1, 1, _BT)
    onehot = (col == idx[:, None]).astype(jnp.bfloat16)
    zq = jnp.dot(onehot, cb16_ref[...],
                 preferred_element_type=jnp.float32)
    zq_ref[...] = zq
    diff = zq - zb
    part = jnp.sum(diff * diff)

    @pl.when(i == 0)
    def _():
        loss_ref[...] = jnp.zeros((1, 1), jnp.float32)

    loss_ref[...] += jnp.reshape(part, (1, 1))


@functools.partial(jax.jit, static_argnames=())
def kernel(z, codebook):
    B, C, H, W = z.shape
    T = B * H * W
    nb = T // _BT
    zp = jnp.transpose(z, (0, 2, 3, 1))
    z_flat = zp.reshape(T, _LATENT_DIM)
    c2 = jnp.sum(codebook ** 2, axis=1).reshape(1, _NUM_CODES)
    cb16 = codebook.astype(jnp.bfloat16)

    zq, idx3, loss_sum = pl.pallas_call(
        _vq_block,
        grid=(nb,),
        in_specs=[
            pl.BlockSpec((_BT, _LATENT_DIM), lambda i: (i, 0)),
            pl.BlockSpec((_NUM_CODES, _LATENT_DIM), lambda i: (0, 0)),
            pl.BlockSpec((_NUM_CODES, _LATENT_DIM), lambda i: (0, 0)),
            pl.BlockSpec((1, _NUM_CODES), lambda i: (0, 0)),
        ],
        out_specs=[
            pl.BlockSpec((_BT, _LATENT_DIM), lambda i: (i, 0)),
            pl.BlockSpec((1, 1, _BT), lambda i: (i, 0, 0)),
            pl.BlockSpec((1, 1), lambda i: (0, 0)),
        ],
        out_shape=[
            jax.ShapeDtypeStruct((T, _LATENT_DIM), jnp.float32),
            jax.ShapeDtypeStruct((nb, 1, _BT), jnp.int32),
            jax.ShapeDtypeStruct((1, 1), jnp.float32),
        ],
    )(z_flat, codebook, cb16, c2)

    out = zq.reshape(B, H, W, C).transpose(0, 3, 1, 2)
    indices = idx3.reshape(T)
    loss = (1.0 + _BETA) * (loss_sum[0, 0] / (T * _LATENT_DIM))
    return out, indices, loss


# speedup vs baseline: 1.5832x; 1.1924x over previous
import functools

import jax
import jax.numpy as jnp
from jax.experimental import pallas as pl

_NUM_CODES = 8192
_LATENT_DIM = 64
_BETA = 0.25
_BT = 256


def _vq_block(z_ref, c_ref, cb16_ref, c2_ref, zq_ref, idx_ref, loss_ref):
    i = pl.program_id(0)
    zb = z_ref[...]
    cb = c_ref[...]
    z2 = jnp.sum(zb * zb, axis=1, keepdims=True)
    m2 = jnp.dot(zb + zb, cb.T, preferred_element_type=jnp.float32)
    dist = (z2 + c2_ref[...]) - m2
    pos = jnp.argmin(dist, axis=1).astype(jnp.int32)
    idx = ((127 - (pos & 127)) << 6) + (pos >> 7)
    idx_ref[...] = idx.reshape(1, 1, _BT)
    col = jax.lax.broadcasted_iota(jnp.int32, (_BT, _NUM_CODES), 1)
    onehot = (col == pos[:, None]).astype(jnp.bfloat16)
    zq = jnp.dot(onehot, cb16_ref[...],
                 preferred_element_type=jnp.float32)
    zq_ref[...] = zq
    diff = zq - zb
    part = jnp.sum(diff * diff)

    @pl.when(i == 0)
    def _():
        loss_ref[...] = jnp.zeros((1, 1), jnp.float32)

    loss_ref[...] += jnp.reshape(part, (1, 1))


def _lane_permute(x):
    a = x.shape[0] // _LATENT_DIM
    y = x.reshape(a, _LATENT_DIM, *x.shape[1:])
    y = jnp.flip(y, axis=0)
    y = jnp.swapaxes(y, 0, 1)
    return y.reshape(x.shape)


@functools.partial(jax.jit, static_argnames=())
def kernel(z, codebook):
    B, C, H, W = z.shape
    T = B * H * W
    nb = T // _BT
    zp = jnp.transpose(z, (0, 2, 3, 1))
    z_flat = zp.reshape(T, _LATENT_DIM)
    cb_perm = _lane_permute(codebook)
    c2 = _lane_permute(jnp.sum(codebook ** 2, axis=1)).reshape(1, _NUM_CODES)
    cb16 = cb_perm.astype(jnp.bfloat16)

    zq, idx3, loss_sum = pl.pallas_call(
        _vq_block,
        grid=(nb,),
        in_specs=[
            pl.BlockSpec((_BT, _LATENT_DIM), lambda i: (i, 0)),
            pl.BlockSpec((_NUM_CODES, _LATENT_DIM), lambda i: (0, 0)),
            pl.BlockSpec((_NUM_CODES, _LATENT_DIM), lambda i: (0, 0)),
            pl.BlockSpec((1, _NUM_CODES), lambda i: (0, 0)),
        ],
        out_specs=[
            pl.BlockSpec((_BT, _LATENT_DIM), lambda i: (i, 0)),
            pl.BlockSpec((1, 1, _BT), lambda i: (i, 0, 0)),
            pl.BlockSpec((1, 1), lambda i: (0, 0)),
        ],
        out_shape=[
            jax.ShapeDtypeStruct((T, _LATENT_DIM), jnp.float32),
            jax.ShapeDtypeStruct((nb, 1, _BT), jnp.int32),
            jax.ShapeDtypeStruct((1, 1), jnp.float32),
        ],
    )(z_flat, cb_perm, cb16, c2)

    out = zq.reshape(B, H, W, C).transpose(0, 3, 1, 2)
    indices = idx3.reshape(T)
    loss = (1.0 + _BETA) * (loss_sum[0, 0] / (T * _LATENT_DIM))
    return out, indices, loss


# speedup vs baseline: 1.6359x; 1.0333x over previous
import functools

import jax
import jax.numpy as jnp
from jax.experimental import pallas as pl
from jax.experimental.pallas import tpu as pltpu, tpu_sc as plsc

_NUM_CODES = 8192
_LATENT_DIM = 64
_BETA = 0.25
_BT = 256
_NW = 32


def _vq_block(z_ref, c_ref, c2_ref, idx_ref, loss_ref):
    i = pl.program_id(0)
    zb = z_ref[...]
    cb = c_ref[...]
    z2 = jnp.sum(zb * zb, axis=1, keepdims=True)
    m2 = jnp.dot(zb + zb, cb.T, preferred_element_type=jnp.float32)
    dist = (z2 + c2_ref[...]) - m2
    pos = jnp.argmin(dist, axis=1).astype(jnp.int32)
    idx = ((127 - (pos & 127)) << 6) + (pos >> 7)
    idx_ref[...] = idx.reshape(1, 1, _BT)
    part = jnp.sum(jnp.min(dist, axis=1))

    @pl.when(i == 0)
    def _():
        loss_ref[...] = jnp.zeros((1, 1), jnp.float32)

    loss_ref[...] += jnp.reshape(part, (1, 1))


def _gather_rows(table_hbm, idx_hbm, out_hbm, idx_v, rows_v, sem):
    bpw = idx_v.shape[0]
    wid = jax.lax.axis_index("s") * 2 + jax.lax.axis_index("c")
    base = wid * bpw
    pltpu.sync_copy(idx_hbm.at[pl.ds(base, bpw)], idx_v)
    pltpu.async_copy(table_hbm.at[idx_v], rows_v, sem).wait()
    pltpu.sync_copy(rows_v, out_hbm.at[pl.ds(base, bpw)])


def _lane_permute(x):
    a = x.shape[0] // _LATENT_DIM
    y = x.reshape(a, _LATENT_DIM, *x.shape[1:])
    y = jnp.flip(y, axis=0)
    y = jnp.swapaxes(y, 0, 1)
    return y.reshape(x.shape)


@functools.partial(jax.jit, static_argnames=())
def kernel(z, codebook):
    B, C, H, W = z.shape
    T = B * H * W
    nb = T // _BT
    zp = jnp.transpose(z, (0, 2, 3, 1))
    z_flat = zp.reshape(T, _LATENT_DIM)
    cb_perm = _lane_permute(codebook)
    c2 = _lane_permute(jnp.sum(codebook ** 2, axis=1)).reshape(1, _NUM_CODES)

    idx3, loss_sum = pl.pallas_call(
        _vq_block,
        grid=(nb,),
        in_specs=[
            pl.BlockSpec((_BT, _LATENT_DIM), lambda i: (i, 0)),
            pl.BlockSpec((_NUM_CODES, _LATENT_DIM), lambda i: (0, 0)),
            pl.BlockSpec((1, _NUM_CODES), lambda i: (0, 0)),
        ],
        out_specs=[
            pl.BlockSpec((1, 1, _BT), lambda i: (i, 0, 0)),
            pl.BlockSpec((1, 1), lambda i: (0, 0)),
        ],
        out_shape=[
            jax.ShapeDtypeStruct((nb, 1, _BT), jnp.int32),
            jax.ShapeDtypeStruct((1, 1), jnp.float32),
        ],
    )(z_flat, cb_perm, c2)

    indices = idx3.reshape(T)

    table = jnp.pad(codebook, ((0, 0), (0, 128 - _LATENT_DIM)))
    bpw = T // _NW
    zq_pad = functools.partial(
        pl.kernel,
        mesh=plsc.VectorSubcoreMesh(core_axis_name="c", subcore_axis_name="s"),
        out_type=jax.ShapeDtypeStruct((T, 128), jnp.float32),
        scratch_types=[
            pltpu.VMEM((bpw,), jnp.int32),
            pltpu.VMEM((bpw, 128), jnp.float32),
            pltpu.SemaphoreType.DMA,
        ],
    )(_gather_rows)(table, indices)
    zq = zq_pad[:, :_LATENT_DIM]

    out = zq.reshape(B, H, W, C).transpose(0, 3, 1, 2)
    loss = (1.0 + _BETA) * (loss_sum[0, 0] / (T * _LATENT_DIM))
    return out, indices, loss
